# baseline (speedup 1.0000x reference)
import functools

import jax
import jax.numpy as jnp
from jax import lax
from jax.experimental import pallas as pl
from jax.experimental.pallas import tpu as pltpu
from jax.experimental.pallas import tpu_sc as plsc

B, N, CIN, C, K = 4, 4096, 3, 128, 32
C2 = 2 * C
M1 = float(B * N)
M2 = float(B * N * K)
RB = N * K
RT = 1024
NT = N // RT
GT = 2048
NGB = RB // GT
NEG = float("-inf")


def _l1_body(x8_ref, w1a_ref, g1a_ref, b1a_ref, w1b_ref, g1b_ref, b1b_ref,
             h_ref, hsq_ref):
    def bn_relu(y, g, b):
        mean = jnp.sum(y, axis=0, keepdims=True) / M1
        var = jnp.sum(y * y, axis=0, keepdims=True) / M1 - mean * mean
        yh = (y - mean) * lax.rsqrt(var + 1e-5) * g + b
        return jnp.where(yh >= 0, yh, 0.01 * yh)

    y1 = jnp.dot(x8_ref[...], w1a_ref[...], preferred_element_type=jnp.float32)
    h1 = bn_relu(y1, g1a_ref[...], b1a_ref[...])
    y2 = jnp.dot(h1, w1b_ref[...], preferred_element_type=jnp.float32)
    h = bn_relu(y2, g1b_ref[...], b1b_ref[...])
    h_ref[...] = h
    hsq_ref[...] = jnp.sum(h * h, axis=1, keepdims=True)


def _layer1(x, W1a, g1a, b1a, W1b, g1b, b1b):
    x8 = jnp.pad(x.reshape(B * N, CIN), ((0, 0), (0, 8 - CIN)))
    w1a = jnp.pad(W1a.T, ((0, 8 - CIN), (0, 0)))
    return pl.pallas_call(
        _l1_body,
        out_shape=(jax.ShapeDtypeStruct((B * N, C), jnp.float32),
                   jax.ShapeDtypeStruct((B * N, 1), jnp.float32)),
    )(x8, w1a, g1a.reshape(1, C), b1a.reshape(1, C),
      W1b.T, g1b.reshape(1, C), b1b.reshape(1, C))


NSEG = N // 128
TOPL = 6
BIG = 1 << 20


def _knn_body(h_ref, sq_ref, idx_ref, *, boff):
    t = pl.program_id(0)
    hb = h_ref[...]
    tile = h_ref[pl.ds(t * RT, RT), :]
    inner = lax.dot_general(tile, hb, (((1,), (1,)), ((), ())),
                            preferred_element_type=jnp.float32)
    score = 2.0 * inner - sq_ref[...]

    v = [jnp.full((RT, 128), NEG, jnp.float32) for _ in range(TOPL)]
    sg = [jnp.full((RT, 128), NSEG, jnp.int32) for _ in range(TOPL)]
    vb = jnp.full((RT, 128), NEG, jnp.float32)
    for s in range(NSEG):
        c = score[:, s * 128:(s + 1) * 128]
        cs = jnp.full((RT, 128), s, jnp.int32)
        for t2 in range(TOPL):
            bt = c > v[t2]
            v[t2], c = jnp.where(bt, c, v[t2]), jnp.where(bt, v[t2], c)
            sg[t2], cs = jnp.where(bt, cs, sg[t2]), jnp.where(bt, sg[t2], cs)
        vb = jnp.maximum(vb, c)

    lane2 = lax.broadcasted_iota(jnp.int32, (RT, 128), 1)
    cols = []
    for _ in range(K):
        m = jnp.max(v[0], axis=1, keepdims=True)
        flat = sg[0] * 128 + lane2
        j = jnp.min(jnp.where(v[0] == m, flat, jnp.int32(BIG)),
                    axis=1, keepdims=True)
        cols.append(j)
        lwin = jnp.bitwise_and(j, 127)
        hit = lane2 == lwin
        for t2 in range(TOPL - 1):
            v[t2] = jnp.where(hit, v[t2 + 1], v[t2])
            sg[t2] = jnp.where(hit, sg[t2 + 1], sg[t2])
        v[TOPL - 1] = jnp.where(hit, vb, v[TOPL - 1])
        sg[TOPL - 1] = jnp.where(hit, NSEG, sg[TOPL - 1])
        vb = jnp.where(hit, NEG, vb)
    idx_ref[...] = jnp.concatenate(cols, axis=1) + boff


def _knn_flat_body(h_ref, sq_ref, idx_ref, *, boff):
    t = pl.program_id(0)
    hb = h_ref[...]
    tile = h_ref[pl.ds(t * RT, RT), :]
    inner = lax.dot_general(tile, hb, (((1,), (1,)), ((), ())),
                            preferred_element_type=jnp.float32)
    score = 2.0 * inner - sq_ref[...]
    iota = lax.broadcasted_iota(jnp.int32, (RT, N), 1)
    d = score
    cc = []
    for _ in range(K):
        mm = jnp.max(d, axis=1, keepdims=True)
        arg = jnp.min(jnp.where(d == mm, iota, jnp.int32(N)),
                      axis=1, keepdims=True)
        cc.append(arg)
        d = jnp.where(iota == arg, NEG, d)
    idx_ref[...] = jnp.concatenate(cc, axis=1) + boff


def _knn_call(body, hb, sqb, boff):
    return pl.pallas_call(
        functools.partial(body, boff=boff),
        grid=(NT,),
        in_specs=[
            pl.BlockSpec((N, C), lambda t: (0, 0)),
            pl.BlockSpec((1, N), lambda t: (0, 0)),
        ],
        out_specs=pl.BlockSpec((RT, K), lambda t: (t, 0)),
        out_shape=jax.ShapeDtypeStruct((N, K), jnp.int32),
    )(hb, sqb)


def _knn_batch(hb, sqb, b):
    idx = _knn_call(_knn_body, hb, sqb, b * N)
    bad = jnp.any(idx >= (b + 1) * N)
    return lax.cond(bad,
                    lambda: _knn_call(_knn_flat_body, hb, sqb, b * N),
                    lambda: idx)


def _fold_body(h_ref, wd_ref, wcd_ref, a_ref, bc_ref):
    h = h_ref[...]
    a_ref[...] = jnp.dot(h, wd_ref[...], preferred_element_type=jnp.float32)
    bc_ref[...] = jnp.dot(h, wcd_ref[...], preferred_element_type=jnp.float32)


def _fold(h, W2a):
    wd = W2a[:, :C].T
    wcd = (W2a[:, C:] - W2a[:, :C]).T
    return pl.pallas_call(
        _fold_body,
        grid=(16,),
        in_specs=[
            pl.BlockSpec((1024, C), lambda i: (i, 0)),
            pl.BlockSpec((C, C2), lambda i: (0, 0)),
            pl.BlockSpec((C, C2), lambda i: (0, 0)),
        ],
        out_specs=(pl.BlockSpec((1024, C2), lambda i: (i, 0)),
                   pl.BlockSpec((1024, C2), lambda i: (i, 0))),
        out_shape=(jax.ShapeDtypeStruct((B * N, C2), jnp.float32),
                   jax.ShapeDtypeStruct((B * N, C2), jnp.float32)),
    )(h, wd, wcd)


_NC, _NS = 2, 16
NW = _NC * _NS
RPW = RB // NW
CH = 128


def _sc_gather(A, fidx):
    mesh = plsc.VectorSubcoreMesh(core_axis_name="c", subcore_axis_name="s")

    @functools.partial(
        pl.kernel, mesh=mesh,
        out_type=jax.ShapeDtypeStruct((RB, C2), jnp.float32),
        scratch_types=[
            pltpu.VMEM((RPW,), jnp.int32),
            pltpu.VMEM((CH, C2), jnp.float32),
            pltpu.SemaphoreType.DMA,
        ],
    )
    def gather_k(a_hbm, idx_hbm, out_hbm, idx_v, rows_v, sem):
        wid = lax.axis_index("s") * _NC + lax.axis_index("c")
        base = wid * RPW
        pltpu.sync_copy(idx_hbm.at[pl.ds(base, RPW)], idx_v)

        def body(i, _):
            pltpu.async_copy(a_hbm.at[idx_v.at[pl.ds(i * CH, CH)]],
                             rows_v, sem).wait()
            pltpu.sync_copy(rows_v, out_hbm.at[pl.ds(base + i * CH, CH)])
            return 0

        lax.fori_loop(0, RPW // CH, body, 0)

    return gather_k(A, fidx)


def _stats_body(y0_ref, bc_ref, s_ref, ss_ref):
    g = pl.program_id(0)
    bc = bc_ref[...]
    y = y0_ref[...] + jnp.broadcast_to(
        bc[:, None, :], (GT // K, K, C2)).reshape(GT, C2)

    @pl.when(g == 0)
    def _():
        s_ref[...] = jnp.zeros_like(s_ref)
        ss_ref[...] = jnp.zeros_like(ss_ref)

    s_ref[...] += jnp.sum(y, axis=0, keepdims=True)
    ss_ref[...] += jnp.sum(y * y, axis=0, keepdims=True)


def _stats(y0b, Bcb):
    return pl.pallas_call(
        _stats_body,
        grid=(NGB,),
        in_specs=[
            pl.BlockSpec((GT, C2), lambda g: (g, 0)),
            pl.BlockSpec((GT // K, C2), lambda g: (g, 0)),
        ],
        out_specs=(pl.BlockSpec((1, C2), lambda g: (0, 0)),
                   pl.BlockSpec((1, C2), lambda g: (0, 0))),
        out_shape=(jax.ShapeDtypeStruct((1, C2), jnp.float32),
                   jax.ShapeDtypeStruct((1, C2), jnp.float32)),
    )(y0b, Bcb)


def _main_body(y0_ref, bc_ref, sc_ref, sh_ref, w_ref,
               zmx_ref, zmn_ref, s_ref, ss_ref):
    g = pl.program_id(0)
    bc = bc_ref[...]
    y = y0_ref[...] + jnp.broadcast_to(
        bc[:, None, :], (GT // K, K, C2)).reshape(GT, C2)
    yh = y * sc_ref[...] + sh_ref[...]
    f = jnp.where(yh >= 0, yh, 0.01 * yh)
    z = jnp.dot(f, w_ref[...], preferred_element_type=jnp.float32)

    @pl.when(g == 0)
    def _():
        s_ref[...] = jnp.zeros_like(s_ref)
        ss_ref[...] = jnp.zeros_like(ss_ref)

    s_ref[...] += jnp.sum(z, axis=0, keepdims=True)
    ss_ref[...] += jnp.sum(z * z, axis=0, keepdims=True)
    z3 = z.reshape(GT // K, K, C2)
    zmx_ref[...] = jnp.max(z3, axis=1)
    zmn_ref[...] = jnp.min(z3, axis=1)


def _main(y0b, Bcb, scale_a, shift_a, W2bT):
    return pl.pallas_call(
        _main_body,
        grid=(NGB,),
        in_specs=[
            pl.BlockSpec((GT, C2), lambda g: (g, 0)),
            pl.BlockSpec((GT // K, C2), lambda g: (g, 0)),
            pl.BlockSpec((1, C2), lambda g: (0, 0)),
            pl.BlockSpec((1, C2), lambda g: (0, 0)),
            pl.BlockSpec((C2, C2), lambda g: (0, 0)),
        ],
        out_specs=(pl.BlockSpec((GT // K, C2), lambda g: (g, 0)),
                   pl.BlockSpec((GT // K, C2), lambda g: (g, 0)),
                   pl.BlockSpec((1, C2), lambda g: (0, 0)),
                   pl.BlockSpec((1, C2), lambda g: (0, 0))),
        out_shape=(jax.ShapeDtypeStruct((N, C2), jnp.float32),
                   jax.ShapeDtypeStruct((N, C2), jnp.float32),
                   jax.ShapeDtypeStruct((1, C2), jnp.float32),
                   jax.ShapeDtypeStruct((1, C2), jnp.float32)),
    )(y0b, Bcb, scale_a, shift_a, W2bT)


def _fin_body(zmx_ref, zmn_ref, sc_ref, sh_ref, o_ref):
    sc = sc_ref[...]
    zz = jnp.where(sc >= 0, zmx_ref[...], zmn_ref[...])
    yh = zz * sc + sh_ref[...]
    o_ref[...] = jnp.where(yh >= 0, yh, 0.01 * yh)


def _final(zmx, zmn, scale_b, shift_b):
    return pl.pallas_call(
        _fin_body,
        grid=(4,),
        in_specs=[
            pl.BlockSpec((1024, C2), lambda i: (i, 0)),
            pl.BlockSpec((1024, C2), lambda i: (i, 0)),
            pl.BlockSpec((1, C2), lambda i: (0, 0)),
            pl.BlockSpec((1, C2), lambda i: (0, 0)),
        ],
        out_specs=pl.BlockSpec((1024, C2), lambda i: (i, 0)),
        out_shape=jax.ShapeDtypeStruct((N, C2), jnp.float32),
    )(zmx, zmn, scale_b, shift_b)


def kernel(x, W1a, g1a, b1a, W1b, g1b, b1b, W2a, g2a, b2a, W2b, g2b, b2b):
    h, hsq = _layer1(x, W1a, g1a, b1a, W1b, g1b, b1b)
    h3 = h.reshape(B, N, C)
    sqT = hsq.reshape(B, 1, N)
    A, Bc = _fold(h, W2a)
    Bcb = [Bc[b * N:(b + 1) * N] for b in range(B)]

    idxs = [_knn_batch(h3[b], sqT[b], b) for b in range(B)]
    y0s, stats_a = [], []
    for b in range(B):
        y0b = _sc_gather(A, idxs[b].reshape(RB))
        y0s.append(y0b)
        stats_a.append(_stats(y0b, Bcb[b]))

    s_a = sum(s for s, _ in stats_a)
    ss_a = sum(ss for _, ss in stats_a)
    mean_a = s_a / M2
    var_a = ss_a / M2 - mean_a * mean_a
    scale_a = g2a.reshape(1, C2) * lax.rsqrt(var_a + 1e-5)
    shift_a = b2a.reshape(1, C2) - mean_a * scale_a

    W2bT = W2b.T
    parts = [_main(y0s[b], Bcb[b], scale_a, shift_a, W2bT) for b in range(B)]
    s_b = sum(p[2] for p in parts)
    ss_b = sum(p[3] for p in parts)
    mean_b = s_b / M2
    var_b = ss_b / M2 - mean_b * mean_b
    scale_b = g2b.reshape(1, C2) * lax.rsqrt(var_b + 1e-5)
    shift_b = b2b.reshape(1, C2) - mean_b * scale_b

    outs = [_final(parts[b][0], parts[b][1], scale_b, shift_b)
            for b in range(B)]
    return jnp.stack(outs)


# speedup vs baseline: 1.1375x; 1.1375x over previous
import functools

import jax
import jax.numpy as jnp
from jax import lax
from jax.experimental import pallas as pl
from jax.experimental.pallas import tpu as pltpu
from jax.experimental.pallas import tpu_sc as plsc

B, N, CIN, C, K = 4, 4096, 3, 128, 32
C2 = 2 * C
M1 = float(B * N)
M2 = float(B * N * K)
RB = N * K
RT = 512
NT = N // RT
GT = 1024
NGB = RB // GT
NEG = float("-inf")


def _l1_body(x8_ref, w1a_ref, g1a_ref, b1a_ref, w1b_ref, g1b_ref, b1b_ref,
             h_ref, hsq_ref):
    def bn_relu(y, g, b):
        mean = jnp.sum(y, axis=0, keepdims=True) / M1
        var = jnp.sum(y * y, axis=0, keepdims=True) / M1 - mean * mean
        yh = (y - mean) * lax.rsqrt(var + 1e-5) * g + b
        return jnp.where(yh >= 0, yh, 0.01 * yh)

    y1 = jnp.dot(x8_ref[...], w1a_ref[...], preferred_element_type=jnp.float32)
    h1 = bn_relu(y1, g1a_ref[...], b1a_ref[...])
    y2 = jnp.dot(h1, w1b_ref[...], preferred_element_type=jnp.float32)
    h = bn_relu(y2, g1b_ref[...], b1b_ref[...])
    h_ref[...] = h
    hsq_ref[...] = jnp.sum(h * h, axis=1, keepdims=True)


def _layer1(x, W1a, g1a, b1a, W1b, g1b, b1b):
    x8 = jnp.pad(x.reshape(B * N, CIN), ((0, 0), (0, 8 - CIN)))
    w1a = jnp.pad(W1a.T, ((0, 8 - CIN), (0, 0)))
    return pl.pallas_call(
        _l1_body,
        out_shape=(jax.ShapeDtypeStruct((B * N, C), jnp.float32),
                   jax.ShapeDtypeStruct((B * N, 1), jnp.float32)),
    )(x8, w1a, g1a.reshape(1, C), b1a.reshape(1, C),
      W1b.T, g1b.reshape(1, C), b1b.reshape(1, C))


NSEG = N // 128
TOPL = 6
BIG = 1 << 20


def _knn_body(h_ref, sq_ref, idx_ref, *, boff):
    t = pl.program_id(0)
    hb = h_ref[...]
    tile = h_ref[pl.ds(t * RT, RT), :]
    inner = lax.dot_general(tile, hb, (((1,), (1,)), ((), ())),
                            preferred_element_type=jnp.float32)
    score = 2.0 * inner - sq_ref[...]

    v = [jnp.full((RT, 128), NEG, jnp.float32) for _ in range(TOPL)]
    vb = jnp.full((RT, 128), NEG, jnp.float32)
    for s2 in range(NSEG):
        c = score[:, s2 * 128:(s2 + 1) * 128]
        ci = lax.bitcast_convert_type(c, jnp.int32)
        segbits = jnp.where(ci < 0, jnp.int32(s2), jnp.int32(NSEG - 1 - s2))
        cp = lax.bitcast_convert_type((ci & jnp.int32(-32)) | segbits,
                                      jnp.float32)
        for t2 in range(TOPL):
            nv = jnp.maximum(cp, v[t2])
            cp = jnp.minimum(cp, v[t2])
            v[t2] = nv
        vb = jnp.maximum(vb, cp)

    lane2 = lax.broadcasted_iota(jnp.int32, (RT, 128), 1)
    cnt = jnp.zeros((RT, 128), jnp.int32)
    cols = []
    for _ in range(K):
        m = jnp.max(v[0], axis=1, keepdims=True)
        lmin = jnp.min(jnp.where(v[0] == m, lane2, jnp.int32(BIG)),
                       axis=1, keepdims=True)
        mi = lax.bitcast_convert_type(m, jnp.int32)
        low = mi & jnp.int32(31)
        seg = jnp.where(mi < 0, low, jnp.int32(NSEG - 1) - low)
        cols.append(seg * 128 + lmin)
        hit = lane2 == lmin
        for t2 in range(TOPL - 1):
            v[t2] = jnp.where(hit, v[t2 + 1], v[t2])
        v[TOPL - 1] = jnp.where(hit, vb, v[TOPL - 1])
        vb = jnp.where(hit, NEG, vb)
        cnt = cnt + hit.astype(jnp.int32)
    badrow = jnp.max(cnt, axis=1, keepdims=True) > TOPL
    idx = jnp.concatenate(cols, axis=1)
    idx_ref[...] = jnp.where(badrow, jnp.int32(BIG), idx) + boff


def _knn_flat_body(h_ref, sq_ref, idx_ref, *, boff):
    t = pl.program_id(0)
    hb = h_ref[...]
    tile = h_ref[pl.ds(t * RT, RT), :]
    inner = lax.dot_general(tile, hb, (((1,), (1,)), ((), ())),
                            preferred_element_type=jnp.float32)
    score = 2.0 * inner - sq_ref[...]
    iota = lax.broadcasted_iota(jnp.int32, (RT, N), 1)
    d = score
    cc = []
    for _ in range(K):
        mm = jnp.max(d, axis=1, keepdims=True)
        arg = jnp.min(jnp.where(d == mm, iota, jnp.int32(N)),
                      axis=1, keepdims=True)
        cc.append(arg)
        d = jnp.where(iota == arg, NEG, d)
    idx_ref[...] = jnp.concatenate(cc, axis=1) + boff


def _knn_call(body, hb, sqb, boff):
    return pl.pallas_call(
        functools.partial(body, boff=boff),
        grid=(NT,),
        in_specs=[
            pl.BlockSpec((N, C), lambda t: (0, 0)),
            pl.BlockSpec((1, N), lambda t: (0, 0)),
        ],
        out_specs=pl.BlockSpec((RT, K), lambda t: (t, 0)),
        out_shape=jax.ShapeDtypeStruct((N, K), jnp.int32),
    )(hb, sqb)


def _knn_batch(hb, sqb, b):
    idx = _knn_call(_knn_body, hb, sqb, b * N)
    bad = jnp.any(idx >= (b + 1) * N)
    return lax.cond(bad,
                    lambda: _knn_call(_knn_flat_body, hb, sqb, b * N),
                    lambda: idx)


def _fold_body(h_ref, wd_ref, wcd_ref, a_ref, bc_ref):
    h = h_ref[...]
    a_ref[...] = jnp.dot(h, wd_ref[...], preferred_element_type=jnp.float32)
    bc_ref[...] = jnp.dot(h, wcd_ref[...], preferred_element_type=jnp.float32)


def _fold(h, W2a):
    wd = W2a[:, :C].T
    wcd = (W2a[:, C:] - W2a[:, :C]).T
    return pl.pallas_call(
        _fold_body,
        grid=(16,),
        in_specs=[
            pl.BlockSpec((1024, C), lambda i: (i, 0)),
            pl.BlockSpec((C, C2), lambda i: (0, 0)),
            pl.BlockSpec((C, C2), lambda i: (0, 0)),
        ],
        out_specs=(pl.BlockSpec((1024, C2), lambda i: (i, 0)),
                   pl.BlockSpec((1024, C2), lambda i: (i, 0))),
        out_shape=(jax.ShapeDtypeStruct((B * N, C2), jnp.float32),
                   jax.ShapeDtypeStruct((B * N, C2), jnp.float32)),
    )(h, wd, wcd)


_NC, _NS = 2, 16
NW = _NC * _NS
RPW = RB // NW
CH = 128


def _sc_gather(A, fidx):
    mesh = plsc.VectorSubcoreMesh(core_axis_name="c", subcore_axis_name="s")

    @functools.partial(
        pl.kernel, mesh=mesh,
        out_type=jax.ShapeDtypeStruct((RB, C2), jnp.float32),
        scratch_types=[
            pltpu.VMEM((RPW,), jnp.int32),
            pltpu.VMEM((CH, C2), jnp.float32),
            pltpu.SemaphoreType.DMA,
        ],
    )
    def gather_k(a_hbm, idx_hbm, out_hbm, idx_v, rows_v, sem):
        wid = lax.axis_index("s") * _NC + lax.axis_index("c")
        base = wid * RPW
        pltpu.sync_copy(idx_hbm.at[pl.ds(base, RPW)], idx_v)

        def body(i, _):
            pltpu.async_copy(a_hbm.at[idx_v.at[pl.ds(i * CH, CH)]],
                             rows_v, sem).wait()
            pltpu.sync_copy(rows_v, out_hbm.at[pl.ds(base + i * CH, CH)])
            return 0

        lax.fori_loop(0, RPW // CH, body, 0)

    return gather_k(A, fidx)


def _stats_body(y0_ref, bc_ref, s_ref, ss_ref):
    g = pl.program_id(0)
    bc = bc_ref[...]
    y = y0_ref[...] + jnp.broadcast_to(
        bc[:, None, :], (GT // K, K, C2)).reshape(GT, C2)

    @pl.when(g == 0)
    def _():
        s_ref[...] = jnp.zeros_like(s_ref)
        ss_ref[...] = jnp.zeros_like(ss_ref)

    s_ref[...] += jnp.sum(y, axis=0, keepdims=True)
    ss_ref[...] += jnp.sum(y * y, axis=0, keepdims=True)


def _stats(y0b, Bcb):
    return pl.pallas_call(
        _stats_body,
        grid=(NGB,),
        in_specs=[
            pl.BlockSpec((GT, C2), lambda g: (g, 0)),
            pl.BlockSpec((GT // K, C2), lambda g: (g, 0)),
        ],
        out_specs=(pl.BlockSpec((1, C2), lambda g: (0, 0)),
                   pl.BlockSpec((1, C2), lambda g: (0, 0))),
        out_shape=(jax.ShapeDtypeStruct((1, C2), jnp.float32),
                   jax.ShapeDtypeStruct((1, C2), jnp.float32)),
    )(y0b, Bcb)


def _main_body(y0_ref, bc_ref, sc_ref, sh_ref, w_ref,
               zmx_ref, zmn_ref, s_ref, ss_ref):
    g = pl.program_id(0)
    bc = bc_ref[...]
    y = y0_ref[...] + jnp.broadcast_to(
        bc[:, None, :], (GT // K, K, C2)).reshape(GT, C2)
    yh = y * sc_ref[...] + sh_ref[...]
    f = jnp.where(yh >= 0, yh, 0.01 * yh)
    z = jnp.dot(f, w_ref[...], preferred_element_type=jnp.float32)

    @pl.when(g == 0)
    def _():
        s_ref[...] = jnp.zeros_like(s_ref)
        ss_ref[...] = jnp.zeros_like(ss_ref)

    s_ref[...] += jnp.sum(z, axis=0, keepdims=True)
    ss_ref[...] += jnp.sum(z * z, axis=0, keepdims=True)
    z3 = z.reshape(GT // K, K, C2)
    zmx_ref[...] = jnp.max(z3, axis=1)
    zmn_ref[...] = jnp.min(z3, axis=1)


def _main(y0b, Bcb, scale_a, shift_a, W2bT):
    return pl.pallas_call(
        _main_body,
        grid=(NGB,),
        in_specs=[
            pl.BlockSpec((GT, C2), lambda g: (g, 0)),
            pl.BlockSpec((GT // K, C2), lambda g: (g, 0)),
            pl.BlockSpec((1, C2), lambda g: (0, 0)),
            pl.BlockSpec((1, C2), lambda g: (0, 0)),
            pl.BlockSpec((C2, C2), lambda g: (0, 0)),
        ],
        out_specs=(pl.BlockSpec((GT // K, C2), lambda g: (g, 0)),
                   pl.BlockSpec((GT // K, C2), lambda g: (g, 0)),
                   pl.BlockSpec((1, C2), lambda g: (0, 0)),
                   pl.BlockSpec((1, C2), lambda g: (0, 0))),
        out_shape=(jax.ShapeDtypeStruct((N, C2), jnp.float32),
                   jax.ShapeDtypeStruct((N, C2), jnp.float32),
                   jax.ShapeDtypeStruct((1, C2), jnp.float32),
                   jax.ShapeDtypeStruct((1, C2), jnp.float32)),
    )(y0b, Bcb, scale_a, shift_a, W2bT)


def _fin_body(zmx_ref, zmn_ref, sc_ref, sh_ref, o_ref):
    sc = sc_ref[...]
    zz = jnp.where(sc >= 0, zmx_ref[...], zmn_ref[...])
    yh = zz * sc + sh_ref[...]
    o_ref[...] = jnp.where(yh >= 0, yh, 0.01 * yh)


def _final(zmx, zmn, scale_b, shift_b):
    return pl.pallas_call(
        _fin_body,
        grid=(4,),
        in_specs=[
            pl.BlockSpec((1024, C2), lambda i: (i, 0)),
            pl.BlockSpec((1024, C2), lambda i: (i, 0)),
            pl.BlockSpec((1, C2), lambda i: (0, 0)),
            pl.BlockSpec((1, C2), lambda i: (0, 0)),
        ],
        out_specs=pl.BlockSpec((1024, C2), lambda i: (i, 0)),
        out_shape=jax.ShapeDtypeStruct((N, C2), jnp.float32),
    )(zmx, zmn, scale_b, shift_b)


def kernel(x, W1a, g1a, b1a, W1b, g1b, b1b, W2a, g2a, b2a, W2b, g2b, b2b):
    h, hsq = _layer1(x, W1a, g1a, b1a, W1b, g1b, b1b)
    h3 = h.reshape(B, N, C)
    sqT = hsq.reshape(B, 1, N)
    A, Bc = _fold(h, W2a)
    Bcb = [Bc[b * N:(b + 1) * N] for b in range(B)]

    idxs = [_knn_batch(h3[b], sqT[b], b) for b in range(B)]
    y0s, stats_a = [], []
    for b in range(B):
        y0b = _sc_gather(A, idxs[b].reshape(RB))
        y0s.append(y0b)
        stats_a.append(_stats(y0b, Bcb[b]))

    s_a = sum(s for s, _ in stats_a)
    ss_a = sum(ss for _, ss in stats_a)
    mean_a = s_a / M2
    var_a = ss_a / M2 - mean_a * mean_a
    scale_a = g2a.reshape(1, C2) * lax.rsqrt(var_a + 1e-5)
    shift_a = b2a.reshape(1, C2) - mean_a * scale_a

    W2bT = W2b.T
    parts = [_main(y0s[b], Bcb[b], scale_a, shift_a, W2bT) for b in range(B)]
    s_b = sum(p[2] for p in parts)
    ss_b = sum(p[3] for p in parts)
    mean_b = s_b / M2
    var_b = ss_b / M2 - mean_b * mean_b
    scale_b = g2b.reshape(1, C2) * lax.rsqrt(var_b + 1e-5)
    shift_b = b2b.reshape(1, C2) - mean_b * scale_b

    outs = [_final(parts[b][0], parts[b][1], scale_b, shift_b)
            for b in range(B)]
    return jnp.stack(outs)


# speedup vs baseline: 1.3349x; 1.1735x over previous
import functools

import jax
import jax.numpy as jnp
from jax import lax
from jax.experimental import pallas as pl
from jax.experimental.pallas import tpu as pltpu
from jax.experimental.pallas import tpu_sc as plsc

B, N, CIN, C, K = 4, 4096, 3, 128, 32
C2 = 2 * C
M1 = float(B * N)
M2 = float(B * N * K)
RB = N * K
RT = 512
NT = N // RT
GT = 2048
NGB = RB // GT
NEG = float("-inf")


def _l1_body(x8_ref, w1a_ref, g1a_ref, b1a_ref, w1b_ref, g1b_ref, b1b_ref,
             h_ref, hsq_ref):
    def bn_relu(y, g, b):
        mean = jnp.sum(y, axis=0, keepdims=True) / M1
        var = jnp.sum(y * y, axis=0, keepdims=True) / M1 - mean * mean
        yh = (y - mean) * lax.rsqrt(var + 1e-5) * g + b
        return jnp.where(yh >= 0, yh, 0.01 * yh)

    y1 = jnp.dot(x8_ref[...], w1a_ref[...], preferred_element_type=jnp.float32)
    h1 = bn_relu(y1, g1a_ref[...], b1a_ref[...])
    y2 = jnp.dot(h1, w1b_ref[...], preferred_element_type=jnp.float32)
    h = bn_relu(y2, g1b_ref[...], b1b_ref[...])
    h_ref[...] = h
    hsq_ref[...] = jnp.sum(h * h, axis=1, keepdims=True)


def _layer1(x, W1a, g1a, b1a, W1b, g1b, b1b):
    x8 = jnp.pad(x.reshape(B * N, CIN), ((0, 0), (0, 8 - CIN)))
    w1a = jnp.pad(W1a.T, ((0, 8 - CIN), (0, 0)))
    return pl.pallas_call(
        _l1_body,
        out_shape=(jax.ShapeDtypeStruct((B * N, C), jnp.float32),
                   jax.ShapeDtypeStruct((B * N, 1), jnp.float32)),
    )(x8, w1a, g1a.reshape(1, C), b1a.reshape(1, C),
      W1b.T, g1b.reshape(1, C), b1b.reshape(1, C))


NSEG = N // 128
TOPL = 6
BIG = 1 << 20


def _knn_body(h_ref, sq_ref, idx_ref, *, boff):
    t = pl.program_id(0)
    hb = h_ref[...]
    tile = h_ref[pl.ds(t * RT, RT), :]
    inner = lax.dot_general(tile, hb, (((1,), (1,)), ((), ())),
                            preferred_element_type=jnp.float32)
    score = 2.0 * inner - sq_ref[...]

    v = [jnp.full((RT, 128), NEG, jnp.float32) for _ in range(TOPL)]
    vb = jnp.full((RT, 128), NEG, jnp.float32)
    for s2 in range(NSEG):
        c = score[:, s2 * 128:(s2 + 1) * 128]
        ci = lax.bitcast_convert_type(c, jnp.int32)
        segbits = jnp.where(ci < 0, jnp.int32(s2), jnp.int32(NSEG - 1 - s2))
        cp = lax.bitcast_convert_type((ci & jnp.int32(-32)) | segbits,
                                      jnp.float32)
        for t2 in range(TOPL):
            nv = jnp.maximum(cp, v[t2])
            cp = jnp.minimum(cp, v[t2])
            v[t2] = nv
        vb = jnp.maximum(vb, cp)

    lane2 = lax.broadcasted_iota(jnp.int32, (RT, 128), 1)
    cnt = jnp.zeros((RT, 128), jnp.int32)
    cols = []
    for _ in range(K):
        m = jnp.max(v[0], axis=1, keepdims=True)
        lmin = jnp.min(jnp.where(v[0] == m, lane2, jnp.int32(BIG)),
                       axis=1, keepdims=True)
        mi = lax.bitcast_convert_type(m, jnp.int32)
        low = mi & jnp.int32(31)
        seg = jnp.where(mi < 0, low, jnp.int32(NSEG - 1) - low)
        cols.append(seg * 128 + lmin)
        hit = lane2 == lmin
        for t2 in range(TOPL - 1):
            v[t2] = jnp.where(hit, v[t2 + 1], v[t2])
        v[TOPL - 1] = jnp.where(hit, vb, v[TOPL - 1])
        vb = jnp.where(hit, NEG, vb)
        cnt = cnt + hit.astype(jnp.int32)
    badrow = jnp.max(cnt, axis=1, keepdims=True) > TOPL
    idx = jnp.concatenate(cols, axis=1)
    idx_ref[...] = jnp.where(badrow, jnp.int32(BIG), idx) + boff


def _knn_flat_body(h_ref, sq_ref, idx_ref, *, boff):
    t = pl.program_id(0)
    hb = h_ref[...]
    tile = h_ref[pl.ds(t * RT, RT), :]
    inner = lax.dot_general(tile, hb, (((1,), (1,)), ((), ())),
                            preferred_element_type=jnp.float32)
    score = 2.0 * inner - sq_ref[...]
    iota = lax.broadcasted_iota(jnp.int32, (RT, N), 1)
    d = score
    cc = []
    for _ in range(K):
        mm = jnp.max(d, axis=1, keepdims=True)
        arg = jnp.min(jnp.where(d == mm, iota, jnp.int32(N)),
                      axis=1, keepdims=True)
        cc.append(arg)
        d = jnp.where(iota == arg, NEG, d)
    idx_ref[...] = jnp.concatenate(cc, axis=1) + boff


def _knn_call(body, hb, sqb, boff):
    return pl.pallas_call(
        functools.partial(body, boff=boff),
        grid=(NT,),
        in_specs=[
            pl.BlockSpec((N, C), lambda t: (0, 0)),
            pl.BlockSpec((1, N), lambda t: (0, 0)),
        ],
        out_specs=pl.BlockSpec((RT, K), lambda t: (t, 0)),
        out_shape=jax.ShapeDtypeStruct((N, K), jnp.int32),
    )(hb, sqb)


def _knn_batch(hb, sqb, b):
    idx = _knn_call(_knn_body, hb, sqb, b * N)
    bad = jnp.any(idx >= (b + 1) * N)
    return lax.cond(bad,
                    lambda: _knn_call(_knn_flat_body, hb, sqb, b * N),
                    lambda: idx)


def _fold_body(h_ref, wd_ref, wcd_ref, a_ref, bc_ref):
    h = h_ref[...]
    a_ref[...] = jnp.dot(h, wd_ref[...], preferred_element_type=jnp.float32)
    bc_ref[...] = jnp.dot(h, wcd_ref[...], preferred_element_type=jnp.float32)


def _fold(h, W2a):
    wd = W2a[:, :C].T
    wcd = (W2a[:, C:] - W2a[:, :C]).T
    return pl.pallas_call(
        _fold_body,
        grid=(16,),
        in_specs=[
            pl.BlockSpec((1024, C), lambda i: (i, 0)),
            pl.BlockSpec((C, C2), lambda i: (0, 0)),
            pl.BlockSpec((C, C2), lambda i: (0, 0)),
        ],
        out_specs=(pl.BlockSpec((1024, C2), lambda i: (i, 0)),
                   pl.BlockSpec((1024, C2), lambda i: (i, 0))),
        out_shape=(jax.ShapeDtypeStruct((B * N, C2), jnp.float32),
                   jax.ShapeDtypeStruct((B * N, C2), jnp.float32)),
    )(h, wd, wcd)


_NC, _NS = 2, 16
NW = _NC * _NS
RPW = RB // NW
CH = 128


def _sc_gather(A, fidx):
    mesh = plsc.VectorSubcoreMesh(core_axis_name="c", subcore_axis_name="s")

    @functools.partial(
        pl.kernel, mesh=mesh,
        out_type=jax.ShapeDtypeStruct((RB, C2), jnp.float32),
        scratch_types=[
            pltpu.VMEM((RPW,), jnp.int32),
            pltpu.VMEM((CH, C2), jnp.float32),
            pltpu.SemaphoreType.DMA,
        ],
    )
    def gather_k(a_hbm, idx_hbm, out_hbm, idx_v, rows_v, sem):
        wid = lax.axis_index("s") * _NC + lax.axis_index("c")
        base = wid * RPW
        pltpu.sync_copy(idx_hbm.at[pl.ds(base, RPW)], idx_v)

        def body(i, _):
            pltpu.async_copy(a_hbm.at[idx_v.at[pl.ds(i * CH, CH)]],
                             rows_v, sem).wait()
            pltpu.sync_copy(rows_v, out_hbm.at[pl.ds(base + i * CH, CH)])
            return 0

        lax.fori_loop(0, RPW // CH, body, 0)

    return gather_k(A, fidx)


def _stats_body(y0_ref, bc_ref, s_ref, ss_ref):
    g = pl.program_id(0)
    bc = bc_ref[...]
    y = y0_ref[...] + jnp.broadcast_to(
        bc[:, None, :], (GT // K, K, C2)).reshape(GT, C2)

    @pl.when(g == 0)
    def _():
        s_ref[...] = jnp.zeros_like(s_ref)
        ss_ref[...] = jnp.zeros_like(ss_ref)

    s_ref[...] += jnp.sum(y, axis=0, keepdims=True)
    ss_ref[...] += jnp.sum(y * y, axis=0, keepdims=True)


def _stats(y0b, Bcb):
    return pl.pallas_call(
        _stats_body,
        grid=(NGB,),
        in_specs=[
            pl.BlockSpec((GT, C2), lambda g: (g, 0)),
            pl.BlockSpec((GT // K, C2), lambda g: (g, 0)),
        ],
        out_specs=(pl.BlockSpec((1, C2), lambda g: (0, 0)),
                   pl.BlockSpec((1, C2), lambda g: (0, 0))),
        out_shape=(jax.ShapeDtypeStruct((1, C2), jnp.float32),
                   jax.ShapeDtypeStruct((1, C2), jnp.float32)),
    )(y0b, Bcb)


def _main_body(y0_ref, bc_ref, sc_ref, sh_ref, w_ref,
               zmx_ref, zmn_ref, s_ref, ss_ref):
    g = pl.program_id(0)
    bc = bc_ref[...]
    y = y0_ref[...] + jnp.broadcast_to(
        bc[:, None, :], (GT // K, K, C2)).reshape(GT, C2)
    yh = y * sc_ref[...] + sh_ref[...]
    f = jnp.where(yh >= 0, yh, 0.01 * yh)
    z = jnp.dot(f, w_ref[...], preferred_element_type=jnp.float32)

    @pl.when(g == 0)
    def _():
        s_ref[...] = jnp.zeros_like(s_ref)
        ss_ref[...] = jnp.zeros_like(ss_ref)

    s_ref[...] += jnp.sum(z, axis=0, keepdims=True)
    ss_ref[...] += jnp.sum(z * z, axis=0, keepdims=True)
    z3 = z.reshape(GT // K, K, C2)
    zmx_ref[...] = jnp.max(z3, axis=1)
    zmn_ref[...] = jnp.min(z3, axis=1)


def _main(y0b, Bcb, scale_a, shift_a, W2bT):
    return pl.pallas_call(
        _main_body,
        grid=(NGB,),
        in_specs=[
            pl.BlockSpec((GT, C2), lambda g: (g, 0)),
            pl.BlockSpec((GT // K, C2), lambda g: (g, 0)),
            pl.BlockSpec((1, C2), lambda g: (0, 0)),
            pl.BlockSpec((1, C2), lambda g: (0, 0)),
            pl.BlockSpec((C2, C2), lambda g: (0, 0)),
        ],
        out_specs=(pl.BlockSpec((GT // K, C2), lambda g: (g, 0)),
                   pl.BlockSpec((GT // K, C2), lambda g: (g, 0)),
                   pl.BlockSpec((1, C2), lambda g: (0, 0)),
                   pl.BlockSpec((1, C2), lambda g: (0, 0))),
        out_shape=(jax.ShapeDtypeStruct((N, C2), jnp.float32),
                   jax.ShapeDtypeStruct((N, C2), jnp.float32),
                   jax.ShapeDtypeStruct((1, C2), jnp.float32),
                   jax.ShapeDtypeStruct((1, C2), jnp.float32)),
    )(y0b, Bcb, scale_a, shift_a, W2bT)


def _fin_body(zmx_ref, zmn_ref, sc_ref, sh_ref, o_ref):
    sc = sc_ref[...]
    zz = jnp.where(sc >= 0, zmx_ref[...], zmn_ref[...])
    yh = zz * sc + sh_ref[...]
    o_ref[...] = jnp.where(yh >= 0, yh, 0.01 * yh)


def _final(zmx, zmn, scale_b, shift_b):
    return pl.pallas_call(
        _fin_body,
        grid=(4,),
        in_specs=[
            pl.BlockSpec((1024, C2), lambda i: (i, 0)),
            pl.BlockSpec((1024, C2), lambda i: (i, 0)),
            pl.BlockSpec((1, C2), lambda i: (0, 0)),
            pl.BlockSpec((1, C2), lambda i: (0, 0)),
        ],
        out_specs=pl.BlockSpec((1024, C2), lambda i: (i, 0)),
        out_shape=jax.ShapeDtypeStruct((N, C2), jnp.float32),
    )(zmx, zmn, scale_b, shift_b)


def kernel(x, W1a, g1a, b1a, W1b, g1b, b1b, W2a, g2a, b2a, W2b, g2b, b2b):
    h, hsq = _layer1(x, W1a, g1a, b1a, W1b, g1b, b1b)
    h3 = h.reshape(B, N, C)
    sqT = hsq.reshape(B, 1, N)
    A, Bc = _fold(h, W2a)
    Bcb = [Bc[b * N:(b + 1) * N] for b in range(B)]

    idxs = [_knn_batch(h3[b], sqT[b], b) for b in range(B)]
    y0s, stats_a = [], []
    for b in range(B):
        y0b = _sc_gather(A, idxs[b].reshape(RB))
        y0s.append(y0b)
        stats_a.append(_stats(y0b, Bcb[b]))

    s_a = sum(s for s, _ in stats_a)
    ss_a = sum(ss for _, ss in stats_a)
    mean_a = s_a / M2
    var_a = ss_a / M2 - mean_a * mean_a
    scale_a = g2a.reshape(1, C2) * lax.rsqrt(var_a + 1e-5)
    shift_a = b2a.reshape(1, C2) - mean_a * scale_a

    W2bT = W2b.T
    parts = [_main(y0s[b], Bcb[b], scale_a, shift_a, W2bT) for b in range(B)]
    s_b = sum(p[2] for p in parts)
    ss_b = sum(p[3] for p in parts)
    mean_b = s_b / M2
    var_b = ss_b / M2 - mean_b * mean_b
    scale_b = g2b.reshape(1, C2) * lax.rsqrt(var_b + 1e-5)
    shift_b = b2b.reshape(1, C2) - mean_b * scale_b

    outs = [_final(parts[b][0], parts[b][1], scale_b, shift_b)
            for b in range(B)]
    return jnp.stack(outs)


# speedup vs baseline: 1.4871x; 1.1140x over previous
import functools

import jax
import jax.numpy as jnp
from jax import lax
from jax.experimental import pallas as pl
from jax.experimental.pallas import tpu as pltpu
from jax.experimental.pallas import tpu_sc as plsc

B, N, CIN, C, K = 4, 4096, 3, 128, 32
C2 = 2 * C
M1 = float(B * N)
M2 = float(B * N * K)
RB = N * K
RT = 512
NT = N // RT
GT = 8192
NGB = RB // GT
NEG = float("-inf")


def _l1_body(x8_ref, w1a_ref, g1a_ref, b1a_ref, w1b_ref, g1b_ref, b1b_ref,
             h_ref, hsq_ref):
    def bn_relu(y, g, b):
        mean = jnp.sum(y, axis=0, keepdims=True) / M1
        var = jnp.sum(y * y, axis=0, keepdims=True) / M1 - mean * mean
        yh = (y - mean) * lax.rsqrt(var + 1e-5) * g + b
        return jnp.where(yh >= 0, yh, 0.01 * yh)

    y1 = jnp.dot(x8_ref[...], w1a_ref[...], preferred_element_type=jnp.float32)
    h1 = bn_relu(y1, g1a_ref[...], b1a_ref[...])
    y2 = jnp.dot(h1, w1b_ref[...], preferred_element_type=jnp.float32)
    h = bn_relu(y2, g1b_ref[...], b1b_ref[...])
    h_ref[...] = h
    hsq_ref[...] = jnp.sum(h * h, axis=1, keepdims=True)


def _layer1(x, W1a, g1a, b1a, W1b, g1b, b1b):
    x8 = jnp.pad(x.reshape(B * N, CIN), ((0, 0), (0, 8 - CIN)))
    w1a = jnp.pad(W1a.T, ((0, 8 - CIN), (0, 0)))
    return pl.pallas_call(
        _l1_body,
        out_shape=(jax.ShapeDtypeStruct((B * N, C), jnp.float32),
                   jax.ShapeDtypeStruct((B * N, 1), jnp.float32)),
    )(x8, w1a, g1a.reshape(1, C), b1a.reshape(1, C),
      W1b.T, g1b.reshape(1, C), b1b.reshape(1, C))


NSEG = N // 128
TOPL = 6
BIG = 1 << 20


def _knn_body(h_ref, sq_ref, idx_ref, *, boff):
    t = pl.program_id(0)
    hb = h_ref[...]
    tile = h_ref[pl.ds(t * RT, RT), :]
    inner = lax.dot_general(tile, hb, (((1,), (1,)), ((), ())),
                            preferred_element_type=jnp.float32)
    score = 2.0 * inner - sq_ref[...]

    v = [jnp.full((RT, 128), NEG, jnp.float32) for _ in range(TOPL)]
    vb = jnp.full((RT, 128), NEG, jnp.float32)
    for s2 in range(NSEG):
        c = score[:, s2 * 128:(s2 + 1) * 128]
        ci = lax.bitcast_convert_type(c, jnp.int32)
        segbits = jnp.where(ci < 0, jnp.int32(s2), jnp.int32(NSEG - 1 - s2))
        cp = lax.bitcast_convert_type((ci & jnp.int32(-32)) | segbits,
                                      jnp.float32)
        for t2 in range(TOPL):
            nv = jnp.maximum(cp, v[t2])
            cp = jnp.minimum(cp, v[t2])
            v[t2] = nv
        vb = jnp.maximum(vb, cp)

    lane2 = lax.broadcasted_iota(jnp.int32, (RT, 128), 1)
    cnt = jnp.zeros((RT, 128), jnp.int32)
    cols = []
    for _ in range(K):
        m = jnp.max(v[0], axis=1, keepdims=True)
        lmin = jnp.min(jnp.where(v[0] == m, lane2, jnp.int32(BIG)),
                       axis=1, keepdims=True)
        mi = lax.bitcast_convert_type(m, jnp.int32)
        low = mi & jnp.int32(31)
        seg = jnp.where(mi < 0, low, jnp.int32(NSEG - 1) - low)
        cols.append(seg * 128 + lmin)
        hit = lane2 == lmin
        for t2 in range(TOPL - 1):
            v[t2] = jnp.where(hit, v[t2 + 1], v[t2])
        v[TOPL - 1] = jnp.where(hit, vb, v[TOPL - 1])
        vb = jnp.where(hit, NEG, vb)
        cnt = cnt + hit.astype(jnp.int32)
    badrow = jnp.max(cnt, axis=1, keepdims=True) > TOPL
    idx = jnp.concatenate(cols, axis=1)
    idx_ref[...] = jnp.where(badrow, jnp.int32(BIG), idx) + boff


def _knn_flat_body(h_ref, sq_ref, idx_ref, *, boff):
    t = pl.program_id(0)
    hb = h_ref[...]
    tile = h_ref[pl.ds(t * RT, RT), :]
    inner = lax.dot_general(tile, hb, (((1,), (1,)), ((), ())),
                            preferred_element_type=jnp.float32)
    score = 2.0 * inner - sq_ref[...]
    iota = lax.broadcasted_iota(jnp.int32, (RT, N), 1)
    d = score
    cc = []
    for _ in range(K):
        mm = jnp.max(d, axis=1, keepdims=True)
        arg = jnp.min(jnp.where(d == mm, iota, jnp.int32(N)),
                      axis=1, keepdims=True)
        cc.append(arg)
        d = jnp.where(iota == arg, NEG, d)
    idx_ref[...] = jnp.concatenate(cc, axis=1) + boff


def _knn_call(body, hb, sqb, boff):
    return pl.pallas_call(
        functools.partial(body, boff=boff),
        grid=(NT,),
        in_specs=[
            pl.BlockSpec((N, C), lambda t: (0, 0)),
            pl.BlockSpec((1, N), lambda t: (0, 0)),
        ],
        out_specs=pl.BlockSpec((RT, K), lambda t: (t, 0)),
        out_shape=jax.ShapeDtypeStruct((N, K), jnp.int32),
    )(hb, sqb)


def _knn_batch(hb, sqb, b):
    idx = _knn_call(_knn_body, hb, sqb, b * N)
    bad = jnp.any(idx >= (b + 1) * N)
    return lax.cond(bad,
                    lambda: _knn_call(_knn_flat_body, hb, sqb, b * N),
                    lambda: idx)


def _fold_body(h_ref, wd_ref, wcd_ref, a_ref, bc_ref):
    h = h_ref[...]
    a_ref[...] = jnp.dot(h, wd_ref[...], preferred_element_type=jnp.float32)
    bc_ref[...] = jnp.dot(h, wcd_ref[...], preferred_element_type=jnp.float32)


def _fold(h, W2a):
    wd = W2a[:, :C].T
    wcd = (W2a[:, C:] - W2a[:, :C]).T
    return pl.pallas_call(
        _fold_body,
        grid=(16,),
        in_specs=[
            pl.BlockSpec((1024, C), lambda i: (i, 0)),
            pl.BlockSpec((C, C2), lambda i: (0, 0)),
            pl.BlockSpec((C, C2), lambda i: (0, 0)),
        ],
        out_specs=(pl.BlockSpec((1024, C2), lambda i: (i, 0)),
                   pl.BlockSpec((1024, C2), lambda i: (i, 0))),
        out_shape=(jax.ShapeDtypeStruct((B * N, C2), jnp.float32),
                   jax.ShapeDtypeStruct((B * N, C2), jnp.float32)),
    )(h, wd, wcd)


_NC, _NS = 2, 16
NW = _NC * _NS
RPW = RB // NW
CH = 128


def _sc_gather(A, fidx):
    mesh = plsc.VectorSubcoreMesh(core_axis_name="c", subcore_axis_name="s")

    @functools.partial(
        pl.kernel, mesh=mesh,
        out_type=jax.ShapeDtypeStruct((RB, C2), jnp.float32),
        scratch_types=[
            pltpu.VMEM((RPW,), jnp.int32),
            pltpu.VMEM((CH, C2), jnp.float32),
            pltpu.SemaphoreType.DMA,
        ],
    )
    def gather_k(a_hbm, idx_hbm, out_hbm, idx_v, rows_v, sem):
        wid = lax.axis_index("s") * _NC + lax.axis_index("c")
        base = wid * RPW
        pltpu.sync_copy(idx_hbm.at[pl.ds(base, RPW)], idx_v)

        def body(i, _):
            pltpu.async_copy(a_hbm.at[idx_v.at[pl.ds(i * CH, CH)]],
                             rows_v, sem).wait()
            pltpu.sync_copy(rows_v, out_hbm.at[pl.ds(base + i * CH, CH)])
            return 0

        lax.fori_loop(0, RPW // CH, body, 0)

    return gather_k(A, fidx)


def _stats_body(y0_ref, bc_ref, s_ref, ss_ref):
    g = pl.program_id(0)
    bc = bc_ref[...]
    y = y0_ref[...] + jnp.broadcast_to(
        bc[:, None, :], (GT // K, K, C2)).reshape(GT, C2)

    @pl.when(g == 0)
    def _():
        s_ref[...] = jnp.zeros_like(s_ref)
        ss_ref[...] = jnp.zeros_like(ss_ref)

    s_ref[...] += jnp.sum(y, axis=0, keepdims=True)
    ss_ref[...] += jnp.sum(y * y, axis=0, keepdims=True)


def _stats(y0b, Bcb):
    return pl.pallas_call(
        _stats_body,
        grid=(NGB,),
        in_specs=[
            pl.BlockSpec((GT, C2), lambda g: (g, 0)),
            pl.BlockSpec((GT // K, C2), lambda g: (g, 0)),
        ],
        out_specs=(pl.BlockSpec((1, C2), lambda g: (0, 0)),
                   pl.BlockSpec((1, C2), lambda g: (0, 0))),
        out_shape=(jax.ShapeDtypeStruct((1, C2), jnp.float32),
                   jax.ShapeDtypeStruct((1, C2), jnp.float32)),
    )(y0b, Bcb)


def _main_body(y0_ref, bc_ref, sc_ref, sh_ref, w_ref,
               zmx_ref, zmn_ref, s_ref, ss_ref):
    g = pl.program_id(0)
    bc = bc_ref[...]
    y = y0_ref[...] + jnp.broadcast_to(
        bc[:, None, :], (GT // K, K, C2)).reshape(GT, C2)
    yh = y * sc_ref[...] + sh_ref[...]
    f = jnp.where(yh >= 0, yh, 0.01 * yh)
    z = jnp.dot(f, w_ref[...], preferred_element_type=jnp.float32)

    @pl.when(g == 0)
    def _():
        s_ref[...] = jnp.zeros_like(s_ref)
        ss_ref[...] = jnp.zeros_like(ss_ref)

    s_ref[...] += jnp.sum(z, axis=0, keepdims=True)
    ss_ref[...] += jnp.sum(z * z, axis=0, keepdims=True)
    z3 = z.reshape(GT // K, K, C2)
    zmx_ref[...] = jnp.max(z3, axis=1)
    zmn_ref[...] = jnp.min(z3, axis=1)


def _main(y0b, Bcb, scale_a, shift_a, W2bT):
    return pl.pallas_call(
        _main_body,
        grid=(NGB,),
        in_specs=[
            pl.BlockSpec((GT, C2), lambda g: (g, 0)),
            pl.BlockSpec((GT // K, C2), lambda g: (g, 0)),
            pl.BlockSpec((1, C2), lambda g: (0, 0)),
            pl.BlockSpec((1, C2), lambda g: (0, 0)),
            pl.BlockSpec((C2, C2), lambda g: (0, 0)),
        ],
        out_specs=(pl.BlockSpec((GT // K, C2), lambda g: (g, 0)),
                   pl.BlockSpec((GT // K, C2), lambda g: (g, 0)),
                   pl.BlockSpec((1, C2), lambda g: (0, 0)),
                   pl.BlockSpec((1, C2), lambda g: (0, 0))),
        out_shape=(jax.ShapeDtypeStruct((N, C2), jnp.float32),
                   jax.ShapeDtypeStruct((N, C2), jnp.float32),
                   jax.ShapeDtypeStruct((1, C2), jnp.float32),
                   jax.ShapeDtypeStruct((1, C2), jnp.float32)),
    )(y0b, Bcb, scale_a, shift_a, W2bT)


def _fin_body(zmx_ref, zmn_ref, sc_ref, sh_ref, o_ref):
    sc = sc_ref[...]
    zz = jnp.where(sc >= 0, zmx_ref[...], zmn_ref[...])
    yh = zz * sc + sh_ref[...]
    o_ref[...] = jnp.where(yh >= 0, yh, 0.01 * yh)


def _final(zmx, zmn, scale_b, shift_b):
    return pl.pallas_call(
        _fin_body,
        grid=(4,),
        in_specs=[
            pl.BlockSpec((1024, C2), lambda i: (i, 0)),
            pl.BlockSpec((1024, C2), lambda i: (i, 0)),
            pl.BlockSpec((1, C2), lambda i: (0, 0)),
            pl.BlockSpec((1, C2), lambda i: (0, 0)),
        ],
        out_specs=pl.BlockSpec((1024, C2), lambda i: (i, 0)),
        out_shape=jax.ShapeDtypeStruct((N, C2), jnp.float32),
    )(zmx, zmn, scale_b, shift_b)


def kernel(x, W1a, g1a, b1a, W1b, g1b, b1b, W2a, g2a, b2a, W2b, g2b, b2b):
    h, hsq = _layer1(x, W1a, g1a, b1a, W1b, g1b, b1b)
    h3 = h.reshape(B, N, C)
    sqT = hsq.reshape(B, 1, N)
    A, Bc = _fold(h, W2a)
    Bcb = [Bc[b * N:(b + 1) * N] for b in range(B)]

    idxs = [_knn_batch(h3[b], sqT[b], b) for b in range(B)]
    y0s, stats_a = [], []
    for b in range(B):
        y0b = _sc_gather(A, idxs[b].reshape(RB))
        y0s.append(y0b)
        stats_a.append(_stats(y0b, Bcb[b]))

    s_a = sum(s for s, _ in stats_a)
    ss_a = sum(ss for _, ss in stats_a)
    mean_a = s_a / M2
    var_a = ss_a / M2 - mean_a * mean_a
    scale_a = g2a.reshape(1, C2) * lax.rsqrt(var_a + 1e-5)
    shift_a = b2a.reshape(1, C2) - mean_a * scale_a

    W2bT = W2b.T
    parts = [_main(y0s[b], Bcb[b], scale_a, shift_a, W2bT) for b in range(B)]
    s_b = sum(p[2] for p in parts)
    ss_b = sum(p[3] for p in parts)
    mean_b = s_b / M2
    var_b = ss_b / M2 - mean_b * mean_b
    scale_b = g2b.reshape(1, C2) * lax.rsqrt(var_b + 1e-5)
    shift_b = b2b.reshape(1, C2) - mean_b * scale_b

    outs = [_final(parts[b][0], parts[b][1], scale_b, shift_b)
            for b in range(B)]
    return jnp.stack(outs)
